# Initial kernel scaffold; baseline (speedup 1.0000x reference)
import jax
import jax.numpy as jnp
from jax.experimental import pallas as pl


def kernel(x_paper, x_author, edge_index_ap, edge_index_pa, params):
    raise NotImplementedError("write your pallas kernel here")


# speedup vs baseline: 54.2941x; 54.2941x over previous
import functools

import jax
import jax.numpy as jnp
from jax import lax
from jax.experimental import pallas as pl
from jax.experimental.pallas import tpu as pltpu
from jax.experimental.pallas import tpu_sc as plsc

NT_LIST = ('paper', 'author')
N = 10000
D = 128
NH = 8
HD = 16
E = 640000
NC = 2
NS = 16
CHUNK = 40
EPC = E // NC
EPS_ = EPC // NS
DEN0 = N
NTOT = N + N // 8
NROWS = 704
TAIL0 = NROWS * (NS - 1)

_EPS = 1e-5


def _edge_body(ke_hbm, qp_hbm, ve_hbm, src_hbm, dst_hbm, z128_hbm,
               acc_out,
               shared_acc,
               idx_s, idx_d, idx_dp, idx_d2, krows, qrows, vrows, exb,
               sem1, sem2, sem3):
    c = lax.axis_index("c")
    s = lax.axis_index("s")
    iota = lax.iota(jnp.int32, 16)
    zero16 = jnp.zeros((16,), jnp.float32)

    r0 = s * NROWS

    @pl.when(s < NS - 1)
    def _zero_main():
        pltpu.sync_copy(z128_hbm.at[pl.ds(r0, NROWS)],
                        shared_acc.at[pl.ds(r0, NROWS)])

    @pl.when(s == NS - 1)
    def _zero_tail():
        pltpu.sync_copy(z128_hbm.at[pl.ds(TAIL0, NTOT - TAIL0)],
                        shared_acc.at[pl.ds(TAIL0, NTOT - TAIL0)])

    plsc.subcore_barrier()

    base = c * EPC + s * EPS_

    def chunk_body(i, _):
        e0 = base + i * CHUNK
        pltpu.sync_copy(src_hbm.at[pl.ds(e0, CHUNK)], idx_s)
        pltpu.sync_copy(dst_hbm.at[pl.ds(e0, CHUNK)], idx_d)
        cp1 = pltpu.async_copy(ke_hbm.at[idx_s], krows, sem1)
        cp2 = pltpu.async_copy(qp_hbm.at[idx_d], qrows, sem2)
        cp3 = pltpu.async_copy(ve_hbm.at[idx_s], vrows, sem3)
        for g in (0, 16, 24):
            dv = idx_d[pl.ds(g, 16)]
            idx_dp[pl.ds(g, 16)] = dv
            idx_d2[pl.ds(g, 16)] = DEN0 + jnp.right_shift(dv, 3)
        cp1.wait()
        cp2.wait()
        cp3.wait()

        def edge_body(e, _):
            scores = jnp.zeros((16,), jnp.float32)
            for h in range(NH):
                kv = krows[e, pl.ds(h * HD, HD)]
                qv = qrows[e, pl.ds(h * HD, HD)]
                sh = jnp.sum(kv * qv)
                scores = scores + jnp.where(iota == h, sh, 0.0)
            exv = jnp.exp(scores)
            for h in range(NH):
                exb[e, pl.ds(h * HD, HD)] = zero16
            dvec = idx_dp[pl.ds(e, 16)]
            goff = (dvec[0] & 7) * HD
            exb[e, pl.ds(goff, HD)] = exv
            for h in range(NH):
                vrows[e, pl.ds(h * HD, HD)] = vrows[e, pl.ds(h * HD, HD)] * exv[h]
            return _

        lax.fori_loop(0, CHUNK, edge_body, None)
        pltpu.sync_copy(vrows, shared_acc.at[idx_d], add=True)
        pltpu.sync_copy(exb, shared_acc.at[idx_d2], add=True)
        return _

    lax.fori_loop(0, EPS_ // CHUNK, chunk_body, None)
    plsc.subcore_barrier()

    @pl.when(s < NS - 1)
    def _dump_main():
        pltpu.sync_copy(shared_acc.at[pl.ds(r0, NROWS)],
                        acc_out.at[c, pl.ds(r0, NROWS)])

    @pl.when(s == NS - 1)
    def _dump_tail():
        pltpu.sync_copy(shared_acc.at[pl.ds(TAIL0, NTOT - TAIL0)],
                        acc_out.at[c, pl.ds(TAIL0, NTOT - TAIL0)])


@functools.cache
def _edge_call():
  return pl.kernel(
    _edge_body,
    out_type=jax.ShapeDtypeStruct((NC, NTOT, D), jnp.float32),
    mesh=plsc.VectorSubcoreMesh(
        core_axis_name="c", subcore_axis_name="s", num_cores=NC, num_subcores=NS
    ),
    scratch_types=[
        pltpu.VMEM_SHARED((NTOT, D), jnp.float32),
        pltpu.VMEM((CHUNK,), jnp.int32),
        pltpu.VMEM((CHUNK,), jnp.int32),
        pltpu.VMEM((CHUNK + 16,), jnp.int32),
        pltpu.VMEM((CHUNK,), jnp.int32),
        pltpu.VMEM((CHUNK, D), jnp.float32),
        pltpu.VMEM((CHUNK, D), jnp.float32),
        pltpu.VMEM((CHUNK, D), jnp.float32),
        pltpu.VMEM((CHUNK, D), jnp.float32),
        pltpu.SemaphoreType.DMA,
        pltpu.SemaphoreType.DMA,
        pltpu.SemaphoreType.DMA,
    ],
    compiler_params=pltpu.CompilerParams(needs_layout_passes=False),
  )


def _ln2(x, g, b):
    mu = jnp.mean(x, axis=-1, keepdims=True)
    var = jnp.mean((x - mu) ** 2, axis=-1, keepdims=True)
    return (x - mu) * lax.rsqrt(var + _EPS) * g + b


def _pre_body(x, qg, qb, qW, qbl, kg, kb, kW, kbl, vg, vb, vW, vbl,
              relk, relv, rscale, tb, tW, tbl, gg, gb, gW, gbl,
              qp_o, ke_o, ve_o, misc_o):
    xin = x[...]
    scale = (1.0 / (1.0 + jnp.exp(-rscale[0, 0])) + 1.0) / (HD ** 0.5)
    q = _ln2(xin, qg[...], qb[...]) @ qW[...] + qbl[...]
    qp_o[...] = q * scale
    ke_o[...] = _ln2(xin, kg[...], kb[...]) @ kW[...] + kbl[...] + relk[...]
    ve_o[...] = _ln2(xin, vg[...], vb[...]) @ vW[...] + vbl[...] + relv[...]
    topo_row = tb[...] @ tW[...] + tbl[...]
    xmean = jnp.mean(xin, axis=0, keepdims=True)
    gctx_row = _ln2(xmean, gg[...], gb[...]) @ gW[...] + gbl[...]
    misc_o[...] = jnp.concatenate([topo_row, gctx_row], axis=0)


_pre_call = pl.pallas_call(
    _pre_body,
    out_shape=(
        jax.ShapeDtypeStruct((N, D), jnp.float32),
        jax.ShapeDtypeStruct((N, D), jnp.float32),
        jax.ShapeDtypeStruct((N, D), jnp.float32),
        jax.ShapeDtypeStruct((2, D), jnp.float32),
    ),
)


BLK = 2000


def _post_body(x, a0, a1, d0, d1, misc,
               og, ob, oW, obl, Wm, bm, W1g, b1g, W2g, b2g,
               fg, fb, fW1, fb1, fW2, fb2, n1g, n1b, n2g, n2b,
               out_o):
    xin = x[...]
    acc = a0[...] + a1[...]
    den = d0[...] + d1[...]
    cols = []
    for h in range(NH):
        dh = den[:, h:h + 1]
        cols.append(jnp.where(dh > 0.0, acc[:, h * HD:(h + 1) * HD] / dh, 0.0))
    local = jnp.concatenate(cols, axis=1)
    loc = _ln2(local, og[...], ob[...]) @ oW[...] + obl[...]
    meta = 1.0 / (1.0 + jnp.exp(-(xin @ Wm[...] + bm[...])))
    topo = jnp.broadcast_to(misc[0:1, :], (BLK, D))
    gctx = jnp.broadcast_to(misc[1:2, :], (BLK, D))
    cat = jnp.concatenate([loc, meta, topo, gctx], axis=1)
    glog = jnp.maximum(cat @ W1g[...] + b1g[...], 0.0) @ W2g[...] + b2g[...]
    gm = jnp.max(glog, axis=-1, keepdims=True)
    ge = jnp.exp(glog - gm)
    gate = ge / jnp.sum(ge, axis=-1, keepdims=True)
    combined = (gate[:, 0:1] * loc + gate[:, 1:2] * meta
                + gate[:, 2:3] * topo + gate[:, 3:4] * gctx)
    hn = _ln2(xin + combined, n1g[...], n1b[...])
    ffin = _ln2(hn, fg[...], fb[...]) @ fW1[...] + fb1[...]
    ffact = 0.5 * ffin * (1.0 + lax.erf(ffin * (2.0 ** -0.5)))
    ff = ffact @ fW2[...] + fb2[...]
    out_o[...] = _ln2(hn + ff, n2g[...], n2b[...])


def _full(shape):
    return pl.BlockSpec(shape, lambda i: tuple(0 for _ in shape))


_post_call = pl.pallas_call(
    _post_body,
    grid=(N // BLK,),
    in_specs=[
        pl.BlockSpec((BLK, D), lambda i: (i, 0)),
        pl.BlockSpec((BLK, D), lambda i: (i, 0)),
        pl.BlockSpec((BLK, D), lambda i: (i, 0)),
        pl.BlockSpec((BLK, 16), lambda i: (i, 0)),
        pl.BlockSpec((BLK, 16), lambda i: (i, 0)),
        _full((2, D)),
        _full((1, D)), _full((1, D)), _full((D, D)), _full((1, D)),
        _full((D, D)), _full((1, D)),
        _full((4 * D, D)), _full((1, D)),
        _full((D, 4)), _full((1, 4)),
        _full((1, D)), _full((1, D)),
        _full((D, 4 * D)), _full((1, 4 * D)),
        _full((4 * D, D)), _full((1, D)),
        _full((1, D)), _full((1, D)),
        _full((1, D)), _full((1, D)),
    ],
    out_specs=pl.BlockSpec((BLK, D), lambda i: (i, 0)),
    out_shape=jax.ShapeDtypeStruct((N, D), jnp.float32),
)


def _row(v):
    return v.reshape(1, -1)


def kernel(x_paper, x_author, edge_index_ap, edge_index_pa, params):
    h = {'paper': x_paper, 'author': x_author}
    rel_of_dst = {'paper': 0, 'author': 1}
    rel_of_src = {'paper': 1, 'author': 0}

    proj = {}
    for nt in NT_LIST:
        p = params[nt]
        rd = rel_of_dst[nt]
        rs = rel_of_src[nt]
        qp, ke, ve, misc = _pre_call(
            h[nt],
            _row(p['q']['g']), _row(p['q']['b']), p['q']['W'], _row(p['q']['bl']),
            _row(p['k']['g']), _row(p['k']['b']), p['k']['W'], _row(p['k']['bl']),
            _row(p['v']['g']), _row(p['v']['b']), p['v']['W'], _row(p['v']['bl']),
            _row(params['rel_k'][rs]), _row(params['rel_v'][rs]),
            params['rel_scale'][rd].reshape(1, 1),
            _row(p['topo']['b']), p['topo']['W'], _row(p['topo']['bl']),
            _row(p['glob']['g']), _row(p['glob']['b']), p['glob']['W'],
            _row(p['glob']['bl']),
        )
        proj[nt] = (qp, ke, ve, misc)

    z128 = jnp.zeros((NTOT, D), jnp.float32)
    edges = {0: edge_index_ap, 1: edge_index_pa}
    agg = {}
    for src_t, dst_t, rid in (('author', 'paper', 0), ('paper', 'author', 1)):
        ei = edges[rid]
        full = _edge_call()(
            proj[src_t][1], proj[dst_t][0], proj[src_t][2],
            ei[0], ei[1], z128,
        )
        acc = full[:, :N, :]
        den = full[:, N:, :].reshape(NC, N, 16)
        agg[dst_t] = (acc, den)

    outs = []
    for nt in NT_LIST:
        p = params[nt]
        acc, den = agg[nt]
        out = _post_call(
            h[nt], acc[0], acc[1], den[0], den[1], proj[nt][3],
            _row(p['out']['g']), _row(p['out']['b']), p['out']['W'],
            _row(p['out']['bl']),
            p['mgate']['W'][:D], _row(p['mgate']['b']),
            p['bg1']['W'], _row(p['bg1']['b']),
            p['bg2']['W'], _row(p['bg2']['b']),
            _row(p['ffn_g']), _row(p['ffn_b']),
            p['ffn_W1'], _row(p['ffn_b1']),
            p['ffn_W2'], _row(p['ffn_b2']),
            _row(p['n1g']), _row(p['n1b']),
            _row(p['n2g']), _row(p['n2b']),
        )
        outs.append(out)
    return jnp.stack(outs, axis=0)


# speedup vs baseline: 54.8150x; 1.0096x over previous
import functools

import jax
import jax.numpy as jnp
from jax import lax
from jax.experimental import pallas as pl
from jax.experimental.pallas import tpu as pltpu
from jax.experimental.pallas import tpu_sc as plsc

NT_LIST = ('paper', 'author')
N = 10000
D = 128
NH = 8
HD = 16
E = 640000
NC = 2
NS = 16
CHUNK = 40
EPC = E // NC
EPS_ = EPC // NS
DEN0 = N
NTOT = N + N // 8
NROWS = 704
TAIL0 = NROWS * (NS - 1)

_EPS = 1e-5


def _edge_body(ke_hbm, qp_hbm, ve_hbm, src_hbm, dst_hbm, z128_hbm,
               acc_out,
               shared_acc,
               idx_s, idx_d, idx_dp, idx_d2, krows, qrows, vrows, exb,
               sem1, sem2, sem3):
    c = lax.axis_index("c")
    s = lax.axis_index("s")
    iota = lax.iota(jnp.int32, 16)
    zero16 = jnp.zeros((16,), jnp.float32)

    r0 = s * NROWS

    @pl.when(s < NS - 1)
    def _zero_main():
        pltpu.sync_copy(z128_hbm.at[pl.ds(r0, NROWS)],
                        shared_acc.at[pl.ds(r0, NROWS)])

    @pl.when(s == NS - 1)
    def _zero_tail():
        pltpu.sync_copy(z128_hbm.at[pl.ds(TAIL0, NTOT - TAIL0)],
                        shared_acc.at[pl.ds(TAIL0, NTOT - TAIL0)])

    plsc.subcore_barrier()

    base = c * EPC + s * EPS_

    def chunk_body(i, _):
        e0 = base + i * CHUNK
        pltpu.sync_copy(src_hbm.at[pl.ds(e0, CHUNK)], idx_s)
        pltpu.sync_copy(dst_hbm.at[pl.ds(e0, CHUNK)], idx_d)
        cp1 = pltpu.async_copy(ke_hbm.at[idx_s], krows, sem1)
        cp2 = pltpu.async_copy(qp_hbm.at[idx_d], qrows, sem2)
        cp3 = pltpu.async_copy(ve_hbm.at[idx_s], vrows, sem3)
        for g in (0, 16, 24):
            dv = idx_d[pl.ds(g, 16)]
            idx_dp[pl.ds(g, 16)] = dv
            idx_d2[pl.ds(g, 16)] = DEN0 + jnp.right_shift(dv, 3)
        cp1.wait()
        cp2.wait()
        cp3.wait()

        def edge_body(e, _):
            scores = jnp.zeros((16,), jnp.float32)
            for h in range(NH):
                kv = krows[e, pl.ds(h * HD, HD)]
                qv = qrows[e, pl.ds(h * HD, HD)]
                sh = jnp.sum(kv * qv)
                scores = scores + jnp.where(iota == h, sh, 0.0)
            exv = jnp.exp(scores)
            for h in range(NH):
                exb[e, pl.ds(h * HD, HD)] = zero16
            dvec = idx_dp[pl.ds(e, 16)]
            goff = (dvec[0] & 7) * HD
            exb[e, pl.ds(goff, HD)] = exv
            for h in range(NH):
                vrows[e, pl.ds(h * HD, HD)] = vrows[e, pl.ds(h * HD, HD)] * exv[h]
            return _

        lax.fori_loop(0, CHUNK, edge_body, None, unroll=4)
        pltpu.sync_copy(vrows, shared_acc.at[idx_d], add=True)
        pltpu.sync_copy(exb, shared_acc.at[idx_d2], add=True)
        return _

    lax.fori_loop(0, EPS_ // CHUNK, chunk_body, None)
    plsc.subcore_barrier()

    @pl.when(s < NS - 1)
    def _dump_main():
        pltpu.sync_copy(shared_acc.at[pl.ds(r0, NROWS)],
                        acc_out.at[c, pl.ds(r0, NROWS)])

    @pl.when(s == NS - 1)
    def _dump_tail():
        pltpu.sync_copy(shared_acc.at[pl.ds(TAIL0, NTOT - TAIL0)],
                        acc_out.at[c, pl.ds(TAIL0, NTOT - TAIL0)])


@functools.cache
def _edge_call():
  return pl.kernel(
    _edge_body,
    out_type=jax.ShapeDtypeStruct((NC, NTOT, D), jnp.float32),
    mesh=plsc.VectorSubcoreMesh(
        core_axis_name="c", subcore_axis_name="s", num_cores=NC, num_subcores=NS
    ),
    scratch_types=[
        pltpu.VMEM_SHARED((NTOT, D), jnp.float32),
        pltpu.VMEM((CHUNK,), jnp.int32),
        pltpu.VMEM((CHUNK,), jnp.int32),
        pltpu.VMEM((CHUNK + 16,), jnp.int32),
        pltpu.VMEM((CHUNK,), jnp.int32),
        pltpu.VMEM((CHUNK, D), jnp.float32),
        pltpu.VMEM((CHUNK, D), jnp.float32),
        pltpu.VMEM((CHUNK, D), jnp.float32),
        pltpu.VMEM((CHUNK, D), jnp.float32),
        pltpu.SemaphoreType.DMA,
        pltpu.SemaphoreType.DMA,
        pltpu.SemaphoreType.DMA,
    ],
    compiler_params=pltpu.CompilerParams(needs_layout_passes=False),
  )


def _ln2(x, g, b):
    mu = jnp.mean(x, axis=-1, keepdims=True)
    var = jnp.mean((x - mu) ** 2, axis=-1, keepdims=True)
    return (x - mu) * lax.rsqrt(var + _EPS) * g + b


def _pre_body(x, qg, qb, qW, qbl, kg, kb, kW, kbl, vg, vb, vW, vbl,
              relk, relv, rscale, tb, tW, tbl, gg, gb, gW, gbl,
              qp_o, ke_o, ve_o, misc_o):
    xin = x[...]
    scale = (1.0 / (1.0 + jnp.exp(-rscale[0, 0])) + 1.0) / (HD ** 0.5)
    q = _ln2(xin, qg[...], qb[...]) @ qW[...] + qbl[...]
    qp_o[...] = q * scale
    ke_o[...] = _ln2(xin, kg[...], kb[...]) @ kW[...] + kbl[...] + relk[...]
    ve_o[...] = _ln2(xin, vg[...], vb[...]) @ vW[...] + vbl[...] + relv[...]
    topo_row = tb[...] @ tW[...] + tbl[...]
    xmean = jnp.mean(xin, axis=0, keepdims=True)
    gctx_row = _ln2(xmean, gg[...], gb[...]) @ gW[...] + gbl[...]
    misc_o[...] = jnp.concatenate([topo_row, gctx_row], axis=0)


_pre_call = pl.pallas_call(
    _pre_body,
    out_shape=(
        jax.ShapeDtypeStruct((N, D), jnp.float32),
        jax.ShapeDtypeStruct((N, D), jnp.float32),
        jax.ShapeDtypeStruct((N, D), jnp.float32),
        jax.ShapeDtypeStruct((2, D), jnp.float32),
    ),
)


BLK = 2000


def _post_body(x, a0, a1, d0, d1, misc,
               og, ob, oW, obl, Wm, bm, W1g, b1g, W2g, b2g,
               fg, fb, fW1, fb1, fW2, fb2, n1g, n1b, n2g, n2b,
               out_o):
    xin = x[...]
    acc = a0[...] + a1[...]
    den = d0[...] + d1[...]
    cols = []
    for h in range(NH):
        dh = den[:, h:h + 1]
        cols.append(jnp.where(dh > 0.0, acc[:, h * HD:(h + 1) * HD] / dh, 0.0))
    local = jnp.concatenate(cols, axis=1)
    loc = _ln2(local, og[...], ob[...]) @ oW[...] + obl[...]
    meta = 1.0 / (1.0 + jnp.exp(-(xin @ Wm[...] + bm[...])))
    topo = jnp.broadcast_to(misc[0:1, :], (BLK, D))
    gctx = jnp.broadcast_to(misc[1:2, :], (BLK, D))
    cat = jnp.concatenate([loc, meta, topo, gctx], axis=1)
    glog = jnp.maximum(cat @ W1g[...] + b1g[...], 0.0) @ W2g[...] + b2g[...]
    gm = jnp.max(glog, axis=-1, keepdims=True)
    ge = jnp.exp(glog - gm)
    gate = ge / jnp.sum(ge, axis=-1, keepdims=True)
    combined = (gate[:, 0:1] * loc + gate[:, 1:2] * meta
                + gate[:, 2:3] * topo + gate[:, 3:4] * gctx)
    hn = _ln2(xin + combined, n1g[...], n1b[...])
    ffin = _ln2(hn, fg[...], fb[...]) @ fW1[...] + fb1[...]
    ffact = 0.5 * ffin * (1.0 + lax.erf(ffin * (2.0 ** -0.5)))
    ff = ffact @ fW2[...] + fb2[...]
    out_o[...] = _ln2(hn + ff, n2g[...], n2b[...])


def _full(shape):
    return pl.BlockSpec(shape, lambda i: tuple(0 for _ in shape))


_post_call = pl.pallas_call(
    _post_body,
    grid=(N // BLK,),
    in_specs=[
        pl.BlockSpec((BLK, D), lambda i: (i, 0)),
        pl.BlockSpec((BLK, D), lambda i: (i, 0)),
        pl.BlockSpec((BLK, D), lambda i: (i, 0)),
        pl.BlockSpec((BLK, 16), lambda i: (i, 0)),
        pl.BlockSpec((BLK, 16), lambda i: (i, 0)),
        _full((2, D)),
        _full((1, D)), _full((1, D)), _full((D, D)), _full((1, D)),
        _full((D, D)), _full((1, D)),
        _full((4 * D, D)), _full((1, D)),
        _full((D, 4)), _full((1, 4)),
        _full((1, D)), _full((1, D)),
        _full((D, 4 * D)), _full((1, 4 * D)),
        _full((4 * D, D)), _full((1, D)),
        _full((1, D)), _full((1, D)),
        _full((1, D)), _full((1, D)),
    ],
    out_specs=pl.BlockSpec((BLK, D), lambda i: (i, 0)),
    out_shape=jax.ShapeDtypeStruct((N, D), jnp.float32),
)


def _row(v):
    return v.reshape(1, -1)


def kernel(x_paper, x_author, edge_index_ap, edge_index_pa, params):
    h = {'paper': x_paper, 'author': x_author}
    rel_of_dst = {'paper': 0, 'author': 1}
    rel_of_src = {'paper': 1, 'author': 0}

    proj = {}
    for nt in NT_LIST:
        p = params[nt]
        rd = rel_of_dst[nt]
        rs = rel_of_src[nt]
        qp, ke, ve, misc = _pre_call(
            h[nt],
            _row(p['q']['g']), _row(p['q']['b']), p['q']['W'], _row(p['q']['bl']),
            _row(p['k']['g']), _row(p['k']['b']), p['k']['W'], _row(p['k']['bl']),
            _row(p['v']['g']), _row(p['v']['b']), p['v']['W'], _row(p['v']['bl']),
            _row(params['rel_k'][rs]), _row(params['rel_v'][rs]),
            params['rel_scale'][rd].reshape(1, 1),
            _row(p['topo']['b']), p['topo']['W'], _row(p['topo']['bl']),
            _row(p['glob']['g']), _row(p['glob']['b']), p['glob']['W'],
            _row(p['glob']['bl']),
        )
        proj[nt] = (qp, ke, ve, misc)

    z128 = jnp.zeros((NTOT, D), jnp.float32)
    edges = {0: edge_index_ap, 1: edge_index_pa}
    agg = {}
    for src_t, dst_t, rid in (('author', 'paper', 0), ('paper', 'author', 1)):
        ei = edges[rid]
        full = _edge_call()(
            proj[src_t][1], proj[dst_t][0], proj[src_t][2],
            ei[0], ei[1], z128,
        )
        acc = full[:, :N, :]
        den = full[:, N:, :].reshape(NC, N, 16)
        agg[dst_t] = (acc, den)

    outs = []
    for nt in NT_LIST:
        p = params[nt]
        acc, den = agg[nt]
        out = _post_call(
            h[nt], acc[0], acc[1], den[0], den[1], proj[nt][3],
            _row(p['out']['g']), _row(p['out']['b']), p['out']['W'],
            _row(p['out']['bl']),
            p['mgate']['W'][:D], _row(p['mgate']['b']),
            p['bg1']['W'], _row(p['bg1']['b']),
            p['bg2']['W'], _row(p['bg2']['b']),
            _row(p['ffn_g']), _row(p['ffn_b']),
            p['ffn_W1'], _row(p['ffn_b1']),
            p['ffn_W2'], _row(p['ffn_b2']),
            _row(p['n1g']), _row(p['n1b']),
            _row(p['n2g']), _row(p['n2b']),
        )
        outs.append(out)
    return jnp.stack(outs, axis=0)
